# baseline (speedup 1.0000x reference)
import functools

import jax
import jax.numpy as jnp
from jax import lax
from jax.experimental import pallas as pl
from jax.experimental.pallas import tpu as pltpu
from jax.experimental.pallas import tpu_sc as plsc

NC = 2
NS = 16
NW = NC * NS
EB = 128


def _mesh():
    return plsc.VectorSubcoreMesh(
        core_axis_name="c", subcore_axis_name="s", num_cores=NC, num_subcores=NS
    )


def _pad_rows(n):
    per = -(-n // NS)
    per = -(-per // 8) * 8
    return per * NS, per


def _sc_params():
    return pltpu.CompilerParams(use_tc_tiling_on_sc=False)


def _make_deg(n, e2):
    assert e2 % (NW * EB) == 0
    chunks = e2 // (NW * EB)
    npad, rows_per_tile = _pad_rows(n)

    @functools.partial(
        pl.kernel,
        out_type=jax.ShapeDtypeStruct((NC, npad, 16), jnp.float32),
        mesh=_mesh(),
        compiler_params=_sc_params(),
        scratch_types=[
            pltpu.VMEM((chunks, EB), jnp.int32),
            pltpu.VMEM((EB, 16), jnp.float32),
            pltpu.VMEM_SHARED((npad, 16), jnp.float32),
            pltpu.SemaphoreType.DMA,
        ],
    )
    def deg_kernel(dsti_hbm, ones_hbm, zeros_hbm, out_hbm, dix, onesv, acc, sem):
        c = lax.axis_index("c")
        s = lax.axis_index("s")
        w = c * NS + s

        pltpu.sync_copy(dsti_hbm.at[pl.ds(w * chunks, chunks)], dix)
        pltpu.sync_copy(ones_hbm, onesv)
        sl = pl.ds(s * rows_per_tile, rows_per_tile)
        pltpu.sync_copy(zeros_hbm, acc.at[sl])
        plsc.subcore_barrier()

        @pl.loop(0, chunks)
        def _edges(i):
            pltpu.async_copy(onesv, acc.at[dix.at[i]], sem, add=True)

        @pl.loop(0, chunks)
        def _drain(i):
            pltpu.make_async_copy(ones_hbm, onesv, sem).wait()

        plsc.subcore_barrier()
        pltpu.sync_copy(acc.at[sl], out_hbm.at[c, sl])

    return deg_kernel


def _make_mp(n, e2, dh2):
    assert e2 % (NS * EB) == 0
    chunks = e2 // (NS * EB)
    nch = e2 // EB
    npad, rows_per_tile = _pad_rows(n)

    nbuf = 2
    assert chunks % nbuf == 0 and chunks >= nbuf

    @functools.partial(
        pl.kernel,
        out_type=jax.ShapeDtypeStruct((NC, npad, dh2), jnp.float32),
        mesh=_mesh(),
        compiler_params=_sc_params(),
        scratch_types=[
            pltpu.VMEM((chunks, 2, EB), jnp.int32),
            [pltpu.VMEM((EB, dh2), jnp.float32) for _ in range(nbuf)],
            pltpu.VMEM_SHARED((npad, dh2), jnp.float32),
            [pltpu.SemaphoreType.DMA for _ in range(nbuf)],
            [pltpu.SemaphoreType.DMA for _ in range(nbuf)],
        ],
    )
    def mp_kernel(g_hbm, idx_hbm, zeros_hbm, out_hbm,
                  idxv, rows, acc, gsems, ssems):
        c = lax.axis_index("c")
        s = lax.axis_index("s")

        pltpu.sync_copy(idx_hbm.at[c, pl.ds(s * chunks, chunks)], idxv)
        sl = pl.ds(s * rows_per_tile, rows_per_tile)
        pltpu.sync_copy(zeros_hbm, acc.at[sl])
        plsc.subcore_barrier()

        for b in range(nbuf - 1):
            pltpu.make_async_copy(g_hbm.at[idxv.at[b, 0]], rows[b], gsems[b]).start()

        @pl.loop(0, chunks, step=nbuf)
        def _edges(i):
            for j in range(nbuf):
                b = j
                nb = (j - 1) % nbuf
                k = i + j

                @pl.when(k >= 1)
                def _drain_prev():
                    pltpu.make_async_copy(
                        g_hbm.at[pl.ds(0, EB)], rows[nb], ssems[nb]).wait()

                @pl.when(k + nbuf - 1 < chunks)
                def _prefetch():
                    pltpu.make_async_copy(
                        g_hbm.at[idxv.at[k + nbuf - 1, 0]], rows[nb],
                        gsems[nb]).start()

                pltpu.make_async_copy(g_hbm.at[pl.ds(0, EB)], rows[b], gsems[b]).wait()
                pltpu.async_copy(rows[b], acc.at[idxv.at[k, 1]], ssems[b], add=True)

        lb = (chunks - 1) % nbuf
        pltpu.make_async_copy(g_hbm.at[pl.ds(0, EB)], rows[lb], ssems[lb]).wait()

        plsc.subcore_barrier()
        pltpu.sync_copy(acc.at[sl], out_hbm.at[c, sl])

    return mp_kernel


_ROWS = 1000


def _m1_body(degp, x, w1, g1o, dinvo, *, dh2):
    deg = degp[0, :, 0:1] + degp[1, :, 0:1] + 1.0
    dinv = lax.rsqrt(deg)
    h = jnp.dot(x[...], w1[...], preferred_element_type=jnp.float32)
    g = h * dinv
    g1o[0] = g[:, :dh2]
    g1o[1] = g[:, dh2:]
    dinvo[...] = dinv


def _m2_body(p0, p1, g1a, g1b, dinv, b1, w2, g2o, *, dh2):
    a0 = p0[0] + g1a[0]
    a1 = p1[0] + g1b[0]
    a = jnp.concatenate([a0, a1], axis=1) * dinv[...] + b1[...]
    z = jnp.maximum(a, 0.0)
    h2 = jnp.dot(z, w2[...], preferred_element_type=jnp.float32)
    g = h2 * dinv[...]
    g2o[0] = g[:, :dh2]
    g2o[1] = g[:, dh2:]


def _fin_body(q0, q1, g2a, g2b, dinv, b2, outo, *, c):
    o0 = q0[0] + g2a[0]
    o1 = q1[0] + g2b[0]
    o = jnp.concatenate([o0, o1], axis=1) * dinv[...] + b2[...]
    mask = lax.broadcasted_iota(jnp.int32, o.shape, 1) < c
    om = jnp.where(mask, o, -jnp.inf)
    m = jnp.max(om, axis=1, keepdims=True)
    ex = jnp.where(mask, jnp.exp(o - m), 0.0)
    ssum = jnp.sum(ex, axis=1, keepdims=True)
    outo[...] = (o - m - jnp.log(ssum))[:, :c]


def _row_spec(cols):
    return pl.BlockSpec((_ROWS, cols), lambda i: (i, 0))


def _part_spec(core, cols):
    return pl.BlockSpec((1, _ROWS, cols), lambda i, _c=core: (_c, i, 0))


def _full_spec(r, cols):
    return pl.BlockSpec((r, cols), lambda i: (0, 0))


def kernel(x, edge_index, nodes, W1, b1, W2, b2):
    del nodes
    n, d = x.shape
    h = W1.shape[1]
    c = W2.shape[1]
    cpad = 64
    h2c = h // 2
    c2c = cpad // 2
    e = edge_index.shape[1]

    egrain = NS * EB * 4
    e2 = -(-e // egrain) * egrain
    src = jnp.pad(edge_index[0], (0, e2 - e))
    dst = jnp.pad(edge_index[1], (0, e2 - e), constant_values=n)

    npad, rows_per_tile = _pad_rows(n)
    grid = (n // _ROWS,)

    ones16 = jnp.ones((EB, 16), jnp.float32)
    zeros16 = jnp.zeros((rows_per_tile, 16), jnp.float32)
    zeros_h = jnp.zeros((rows_per_tile, h2c), jnp.float32)
    zeros_c = jnp.zeros((rows_per_tile, c2c), jnp.float32)

    src2 = jnp.stack([src, src + n])
    dstb = jnp.broadcast_to(dst, (2, e2))
    idx_mp = (jnp.stack([src2, dstb], axis=1)
              .reshape(2, 2, e2 // EB, EB)
              .transpose(0, 2, 1, 3))

    deg_parts = _make_deg(n, e2)(dst.reshape(e2 // EB, EB), ones16, zeros16)

    g1, dinv = pl.pallas_call(
        functools.partial(_m1_body, dh2=h2c),
        grid=grid,
        in_specs=[pl.BlockSpec((2, _ROWS, 16), lambda i: (0, i, 0)),
                  _row_spec(d), _full_spec(d, h)],
        out_specs=[pl.BlockSpec((2, _ROWS, h2c), lambda i: (0, i, 0)),
                   _row_spec(1)],
        out_shape=[
            jax.ShapeDtypeStruct((2, n, h2c), jnp.float32),
            jax.ShapeDtypeStruct((n, 1), jnp.float32),
        ],
    )(deg_parts, x, W1)

    p = _make_mp(n, e2, h2c)(g1.reshape(2 * n, h2c), idx_mp, zeros_h)

    w2p = jnp.pad(W2, ((0, 0), (0, cpad - c)))
    b1r = b1.reshape(1, h)
    g2 = pl.pallas_call(
        functools.partial(_m2_body, dh2=c2c),
        grid=grid,
        in_specs=[_part_spec(0, h2c), _part_spec(1, h2c),
                  pl.BlockSpec((1, _ROWS, h2c), lambda i: (0, i, 0)),
                  pl.BlockSpec((1, _ROWS, h2c), lambda i: (1, i, 0)),
                  _row_spec(1), _full_spec(1, h), _full_spec(h, cpad)],
        out_specs=pl.BlockSpec((2, _ROWS, c2c), lambda i: (0, i, 0)),
        out_shape=jax.ShapeDtypeStruct((2, n, c2c), jnp.float32),
    )(p, p, g1, g1, dinv, b1r, w2p)

    q = _make_mp(n, e2, c2c)(g2.reshape(2 * n, c2c), idx_mp, zeros_c)

    b2p = jnp.pad(b2, (0, cpad - c)).reshape(1, cpad)
    out = pl.pallas_call(
        functools.partial(_fin_body, c=c),
        grid=grid,
        in_specs=[_part_spec(0, c2c), _part_spec(1, c2c),
                  pl.BlockSpec((1, _ROWS, c2c), lambda i: (0, i, 0)),
                  pl.BlockSpec((1, _ROWS, c2c), lambda i: (1, i, 0)),
                  _row_spec(1), _full_spec(1, cpad)],
        out_specs=_row_spec(c),
        out_shape=jax.ShapeDtypeStruct((n, c), jnp.float32),
    )(q, q, g2, g2, dinv, b2p)
    return out


# speedup vs baseline: 1.3277x; 1.3277x over previous
import functools

import jax
import jax.numpy as jnp
from jax import lax
from jax.experimental import pallas as pl
from jax.experimental.pallas import tpu as pltpu
from jax.experimental.pallas import tpu_sc as plsc

NC = 2
NS = 16
NW = NC * NS
EB = 128


def _mesh():
    return plsc.VectorSubcoreMesh(
        core_axis_name="c", subcore_axis_name="s", num_cores=NC, num_subcores=NS
    )


def _pad_rows(n):
    per = -(-n // NS)
    per = -(-per // 8) * 8
    return per * NS, per


def _sc_params():
    return pltpu.CompilerParams(use_tc_tiling_on_sc=False)


def _make_deg(n, e2):
    assert e2 % (NW * EB) == 0
    chunks = e2 // (NW * EB)
    npad, rows_per_tile = _pad_rows(n)

    @functools.partial(
        pl.kernel,
        out_type=jax.ShapeDtypeStruct((NC, npad, 16), jnp.float32),
        mesh=_mesh(),
        compiler_params=_sc_params(),
        scratch_types=[
            pltpu.VMEM((chunks, EB), jnp.int32),
            pltpu.VMEM((EB, 16), jnp.float32),
            pltpu.VMEM_SHARED((npad, 16), jnp.float32),
            pltpu.SemaphoreType.DMA,
        ],
    )
    def deg_kernel(dsti_hbm, ones_hbm, zeros_hbm, out_hbm, dix, onesv, acc, sem):
        c = lax.axis_index("c")
        s = lax.axis_index("s")
        w = c * NS + s

        pltpu.sync_copy(dsti_hbm.at[pl.ds(w * chunks, chunks)], dix)
        pltpu.sync_copy(ones_hbm, onesv)
        sl = pl.ds(s * rows_per_tile, rows_per_tile)
        pltpu.sync_copy(zeros_hbm, acc.at[sl])
        plsc.subcore_barrier()

        @pl.loop(0, chunks)
        def _edges(i):
            pltpu.async_copy(onesv, acc.at[dix.at[i]], sem, add=True)

        @pl.loop(0, chunks)
        def _drain(i):
            pltpu.make_async_copy(ones_hbm, onesv, sem).wait()

        plsc.subcore_barrier()
        pltpu.sync_copy(acc.at[sl], out_hbm.at[c, sl])

    return deg_kernel


def _make_mp(n, e2, dh2):
    assert e2 % (NS * EB) == 0
    chunks = e2 // (NS * EB)
    nch = e2 // EB
    npad, rows_per_tile = _pad_rows(n)

    @functools.partial(
        pl.kernel,
        out_type=jax.ShapeDtypeStruct((NC, npad, dh2), jnp.float32),
        mesh=_mesh(),
        compiler_params=_sc_params(),
        scratch_types=[
            pltpu.VMEM((chunks, 2, EB), jnp.int32),
            pltpu.VMEM((EB, dh2), jnp.float32),
            pltpu.VMEM((EB, dh2), jnp.float32),
            pltpu.VMEM_SHARED((npad, dh2), jnp.float32),
            pltpu.SemaphoreType.DMA,
            pltpu.SemaphoreType.DMA,
            pltpu.SemaphoreType.DMA,
            pltpu.SemaphoreType.DMA,
        ],
    )
    def mp_kernel(g_hbm, idx_hbm, zeros_hbm, out_hbm,
                  idxv, rows0, rows1, acc, gsem0, gsem1, ssem0, ssem1):
        c = lax.axis_index("c")
        s = lax.axis_index("s")

        pltpu.sync_copy(idx_hbm.at[c, pl.ds(s * chunks, chunks)], idxv)
        sl = pl.ds(s * rows_per_tile, rows_per_tile)
        pltpu.sync_copy(zeros_hbm, acc.at[sl])
        plsc.subcore_barrier()

        bufs = ((rows0, gsem0, ssem0), (rows1, gsem1, ssem1))

        pltpu.make_async_copy(g_hbm.at[idxv.at[0, 0]], rows0, gsem0).start()

        @pl.loop(0, chunks, step=2)
        def _edges(i):
            for j in range(2):
                rb, gb, sb = bufs[j]
                ro, go, so = bufs[1 - j]
                k = i + j

                @pl.when(k >= 1)
                def _drain_other():
                    pltpu.make_async_copy(g_hbm.at[pl.ds(0, EB)], ro, so).wait()

                @pl.when(k + 1 < chunks)
                def _prefetch():
                    pltpu.make_async_copy(
                        g_hbm.at[idxv.at[k + 1, 0]], ro, go).start()

                pltpu.make_async_copy(g_hbm.at[pl.ds(0, EB)], rb, gb).wait()
                pltpu.async_copy(rb, acc.at[idxv.at[k, 1]], sb, add=True)

        pltpu.make_async_copy(
            g_hbm.at[pl.ds(0, EB)], bufs[(chunks - 1) % 2][0],
            bufs[(chunks - 1) % 2][2]).wait()

        plsc.subcore_barrier()
        pltpu.sync_copy(acc.at[sl], out_hbm.at[c, sl])

    return mp_kernel


_ROWS = 1000


def _m1_body(degp, x, w1, g1o, dinvo, *, dh2):
    deg = degp[0, :, 0:1] + degp[1, :, 0:1] + 1.0
    dinv = lax.rsqrt(deg)
    h = jnp.dot(x[...], w1[...], preferred_element_type=jnp.float32)
    g = h * dinv
    g1o[0] = g[:, :dh2]
    g1o[1] = g[:, dh2:]
    dinvo[...] = dinv


def _m2_body(p0, p1, g1a, g1b, dinv, b1, w2, g2o, *, dh2):
    a0 = p0[0] + g1a[0]
    a1 = p1[0] + g1b[0]
    a = jnp.concatenate([a0, a1], axis=1) * dinv[...] + b1[...]
    z = jnp.maximum(a, 0.0)
    h2 = jnp.dot(z, w2[...], preferred_element_type=jnp.float32)
    g = h2 * dinv[...]
    g2o[0] = g[:, :dh2]
    g2o[1] = g[:, dh2:]


def _fin_body(q0, q1, g2a, g2b, dinv, b2, outo, *, c):
    o0 = q0[0] + g2a[0]
    o1 = q1[0] + g2b[0]
    o = jnp.concatenate([o0, o1], axis=1) * dinv[...] + b2[...]
    mask = lax.broadcasted_iota(jnp.int32, o.shape, 1) < c
    om = jnp.where(mask, o, -jnp.inf)
    m = jnp.max(om, axis=1, keepdims=True)
    ex = jnp.where(mask, jnp.exp(o - m), 0.0)
    ssum = jnp.sum(ex, axis=1, keepdims=True)
    outo[...] = (o - m - jnp.log(ssum))[:, :c]


def _row_spec(cols):
    return pl.BlockSpec((_ROWS, cols), lambda i: (i, 0))


def _part_spec(core, cols):
    return pl.BlockSpec((1, _ROWS, cols), lambda i, _c=core: (_c, i, 0))


def _full_spec(r, cols):
    return pl.BlockSpec((r, cols), lambda i: (0, 0))


def kernel(x, edge_index, nodes, W1, b1, W2, b2):
    del nodes
    n, d = x.shape
    h = W1.shape[1]
    c = W2.shape[1]
    cpad = 64
    h2c = h // 2
    c2c = cpad // 2
    e = edge_index.shape[1]

    egrain = NS * EB * 2
    e2 = -(-e // egrain) * egrain
    src = jnp.pad(edge_index[0], (0, e2 - e))
    dst = jnp.pad(edge_index[1], (0, e2 - e), constant_values=n)

    npad, rows_per_tile = _pad_rows(n)
    grid = (n // _ROWS,)

    ones16 = jnp.ones((EB, 16), jnp.float32)
    zeros16 = jnp.zeros((rows_per_tile, 16), jnp.float32)
    zeros_h = jnp.zeros((rows_per_tile, h2c), jnp.float32)
    zeros_c = jnp.zeros((rows_per_tile, c2c), jnp.float32)

    src2 = jnp.stack([src, src + n])
    dstb = jnp.broadcast_to(dst, (2, e2))
    idx_mp = (jnp.stack([src2, dstb], axis=1)
              .reshape(2, 2, e2 // EB, EB)
              .transpose(0, 2, 1, 3))

    deg_parts = _make_deg(n, e2)(dst.reshape(e2 // EB, EB), ones16, zeros16)

    g1, dinv = pl.pallas_call(
        functools.partial(_m1_body, dh2=h2c),
        grid=grid,
        in_specs=[pl.BlockSpec((2, _ROWS, 16), lambda i: (0, i, 0)),
                  _row_spec(d), _full_spec(d, h)],
        out_specs=[pl.BlockSpec((2, _ROWS, h2c), lambda i: (0, i, 0)),
                   _row_spec(1)],
        out_shape=[
            jax.ShapeDtypeStruct((2, n, h2c), jnp.float32),
            jax.ShapeDtypeStruct((n, 1), jnp.float32),
        ],
    )(deg_parts, x, W1)

    p = _make_mp(n, e2, h2c)(g1.reshape(2 * n, h2c), idx_mp, zeros_h)

    w2p = jnp.pad(W2, ((0, 0), (0, cpad - c)))
    b1r = b1.reshape(1, h)
    g2 = pl.pallas_call(
        functools.partial(_m2_body, dh2=c2c),
        grid=grid,
        in_specs=[_part_spec(0, h2c), _part_spec(1, h2c),
                  pl.BlockSpec((1, _ROWS, h2c), lambda i: (0, i, 0)),
                  pl.BlockSpec((1, _ROWS, h2c), lambda i: (1, i, 0)),
                  _row_spec(1), _full_spec(1, h), _full_spec(h, cpad)],
        out_specs=pl.BlockSpec((2, _ROWS, c2c), lambda i: (0, i, 0)),
        out_shape=jax.ShapeDtypeStruct((2, n, c2c), jnp.float32),
    )(p, p, g1, g1, dinv, b1r, w2p)

    q = _make_mp(n, e2, c2c)(g2.reshape(2 * n, c2c), idx_mp, zeros_c)

    b2p = jnp.pad(b2, (0, cpad - c)).reshape(1, cpad)
    out = pl.pallas_call(
        functools.partial(_fin_body, c=c),
        grid=grid,
        in_specs=[_part_spec(0, c2c), _part_spec(1, c2c),
                  pl.BlockSpec((1, _ROWS, c2c), lambda i: (0, i, 0)),
                  pl.BlockSpec((1, _ROWS, c2c), lambda i: (1, i, 0)),
                  _row_spec(1), _full_spec(1, cpad)],
        out_specs=_row_spec(c),
        out_shape=jax.ShapeDtypeStruct((n, c), jnp.float32),
    )(q, q, g2, g2, dinv, b2p)
    return out


# speedup vs baseline: 1.5021x; 1.1314x over previous
import functools

import jax
import jax.numpy as jnp
from jax import lax
from jax.experimental import pallas as pl
from jax.experimental.pallas import tpu as pltpu
from jax.experimental.pallas import tpu_sc as plsc

NC = 2
NS = 16
NW = NC * NS
EB = 128


def _mesh():
    return plsc.VectorSubcoreMesh(
        core_axis_name="c", subcore_axis_name="s", num_cores=NC, num_subcores=NS
    )


def _pad_rows(n):
    per = -(-n // NS)
    per = -(-per // 8) * 8
    return per * NS, per


def _sc_params():
    return pltpu.CompilerParams(use_tc_tiling_on_sc=False)


def _make_deg(n, e2):
    assert e2 % (NW * EB) == 0
    chunks = e2 // (NW * EB)
    npad, rows_per_tile = _pad_rows(n)

    @functools.partial(
        pl.kernel,
        out_type=jax.ShapeDtypeStruct((NC, npad, 16), jnp.float32),
        mesh=_mesh(),
        compiler_params=_sc_params(),
        scratch_types=[
            pltpu.VMEM((chunks, EB), jnp.int32),
            pltpu.VMEM((EB, 16), jnp.float32),
            pltpu.VMEM_SHARED((npad, 16), jnp.float32),
            pltpu.SemaphoreType.DMA,
        ],
    )
    def deg_kernel(dsti_hbm, ones_hbm, zeros_hbm, out_hbm, dix, onesv, acc, sem):
        c = lax.axis_index("c")
        s = lax.axis_index("s")
        w = c * NS + s

        pltpu.sync_copy(dsti_hbm.at[pl.ds(w * chunks, chunks)], dix)
        pltpu.sync_copy(ones_hbm, onesv)
        sl = pl.ds(s * rows_per_tile, rows_per_tile)
        pltpu.sync_copy(zeros_hbm, acc.at[sl])
        plsc.subcore_barrier()

        @pl.loop(0, chunks)
        def _edges(i):
            pltpu.async_copy(onesv, acc.at[dix.at[i]], sem, add=True)

        @pl.loop(0, chunks)
        def _drain(i):
            pltpu.make_async_copy(ones_hbm, onesv, sem).wait()

        plsc.subcore_barrier()
        pltpu.sync_copy(acc.at[sl], out_hbm.at[c, sl])

    return deg_kernel


def _make_mp(n, e2, dh2, spmem_table=False):
    assert e2 % (NS * EB) == 0
    chunks = e2 // (NS * EB)
    nch = e2 // EB
    npad, rows_per_tile = _pad_rows(n)
    stage_rows = n // NS

    scratch = [
        pltpu.VMEM((chunks, 2, EB), jnp.int32),
        pltpu.VMEM((EB, dh2), jnp.float32),
        pltpu.VMEM((EB, dh2), jnp.float32),
        pltpu.VMEM_SHARED((npad, dh2), jnp.float32),
        pltpu.SemaphoreType.DMA,
        pltpu.SemaphoreType.DMA,
        pltpu.SemaphoreType.DMA,
        pltpu.SemaphoreType.DMA,
    ]
    if spmem_table:
        scratch.append(pltpu.VMEM_SHARED((n, dh2), jnp.float32))

    @functools.partial(
        pl.kernel,
        out_type=jax.ShapeDtypeStruct((NC, npad, dh2), jnp.float32),
        mesh=_mesh(),
        compiler_params=_sc_params(),
        scratch_types=scratch,
    )
    def mp_kernel(g_hbm_in, idx_hbm, zeros_hbm, out_hbm,
                  idxv, rows0, rows1, acc, gsem0, gsem1, ssem0, ssem1,
                  *maybe_tbl):
        c = lax.axis_index("c")
        s = lax.axis_index("s")

        if spmem_table:
            g_hbm = maybe_tbl[0]
            ci = 0
            pltpu.sync_copy(
                g_hbm_in.at[pl.ds(c * n + s * stage_rows, stage_rows)],
                g_hbm.at[pl.ds(s * stage_rows, stage_rows)])
        else:
            g_hbm = g_hbm_in
            ci = c

        pltpu.sync_copy(idx_hbm.at[ci, pl.ds(s * chunks, chunks)], idxv)
        sl = pl.ds(s * rows_per_tile, rows_per_tile)
        pltpu.sync_copy(zeros_hbm, acc.at[sl])
        plsc.subcore_barrier()

        bufs = ((rows0, gsem0, ssem0), (rows1, gsem1, ssem1))

        pltpu.make_async_copy(g_hbm.at[idxv.at[0, 0]], rows0, gsem0).start()

        @pl.loop(0, chunks, step=2)
        def _edges(i):
            for j in range(2):
                rb, gb, sb = bufs[j]
                ro, go, so = bufs[1 - j]
                k = i + j

                @pl.when(k >= 1)
                def _drain_other():
                    pltpu.make_async_copy(g_hbm_in.at[pl.ds(0, EB)], ro, so).wait()

                @pl.when(k + 1 < chunks)
                def _prefetch():
                    pltpu.make_async_copy(
                        g_hbm.at[idxv.at[k + 1, 0]], ro, go).start()

                pltpu.make_async_copy(g_hbm_in.at[pl.ds(0, EB)], rb, gb).wait()
                pltpu.async_copy(rb, acc.at[idxv.at[k, 1]], sb, add=True)

        pltpu.make_async_copy(
            g_hbm_in.at[pl.ds(0, EB)], bufs[(chunks - 1) % 2][0],
            bufs[(chunks - 1) % 2][2]).wait()

        plsc.subcore_barrier()
        pltpu.sync_copy(acc.at[sl], out_hbm.at[c, sl])

    return mp_kernel


_ROWS = 1000


def _m1_body(degp, x, w1, g1o, dinvo, *, dh2):
    deg = degp[0, :, 0:1] + degp[1, :, 0:1] + 1.0
    dinv = lax.rsqrt(deg)
    h = jnp.dot(x[...], w1[...], preferred_element_type=jnp.float32)
    g = h * dinv
    g1o[0] = g[:, :dh2]
    g1o[1] = g[:, dh2:]
    dinvo[...] = dinv


def _m2_body(p0, p1, g1a, g1b, dinv, b1, w2, g2o, *, dh2):
    a0 = p0[0] + g1a[0]
    a1 = p1[0] + g1b[0]
    a = jnp.concatenate([a0, a1], axis=1) * dinv[...] + b1[...]
    z = jnp.maximum(a, 0.0)
    h2 = jnp.dot(z, w2[...], preferred_element_type=jnp.float32)
    g = h2 * dinv[...]
    g2o[0] = g[:, :dh2]
    g2o[1] = g[:, dh2:]


def _fin_body(q0, q1, g2a, g2b, dinv, b2, outo, *, c):
    o0 = q0[0] + g2a[0]
    o1 = q1[0] + g2b[0]
    o = jnp.concatenate([o0, o1], axis=1) * dinv[...] + b2[...]
    mask = lax.broadcasted_iota(jnp.int32, o.shape, 1) < c
    om = jnp.where(mask, o, -jnp.inf)
    m = jnp.max(om, axis=1, keepdims=True)
    ex = jnp.where(mask, jnp.exp(o - m), 0.0)
    ssum = jnp.sum(ex, axis=1, keepdims=True)
    outo[...] = (o - m - jnp.log(ssum))[:, :c]


def _row_spec(cols):
    return pl.BlockSpec((_ROWS, cols), lambda i: (i, 0))


def _part_spec(core, cols):
    return pl.BlockSpec((1, _ROWS, cols), lambda i, _c=core: (_c, i, 0))


def _full_spec(r, cols):
    return pl.BlockSpec((r, cols), lambda i: (0, 0))


def kernel(x, edge_index, nodes, W1, b1, W2, b2):
    del nodes
    n, d = x.shape
    h = W1.shape[1]
    c = W2.shape[1]
    cpad = 64
    h2c = h // 2
    c2c = cpad // 2
    e = edge_index.shape[1]

    egrain = NS * EB * 2
    e2 = -(-e // egrain) * egrain
    src = jnp.pad(edge_index[0], (0, e2 - e))
    dst = jnp.pad(edge_index[1], (0, e2 - e), constant_values=n)

    npad, rows_per_tile = _pad_rows(n)
    grid = (n // _ROWS,)

    ones16 = jnp.ones((EB, 16), jnp.float32)
    zeros16 = jnp.zeros((rows_per_tile, 16), jnp.float32)
    zeros_h = jnp.zeros((rows_per_tile, h2c), jnp.float32)
    zeros_c = jnp.zeros((rows_per_tile, c2c), jnp.float32)

    src2 = jnp.stack([src, src + n])
    dstb = jnp.broadcast_to(dst, (2, e2))
    idx_mp = (jnp.stack([src2, dstb], axis=1)
              .reshape(2, 2, e2 // EB, EB)
              .transpose(0, 2, 1, 3))

    deg_parts = _make_deg(n, e2)(dst.reshape(e2 // EB, EB), ones16, zeros16)

    g1, dinv = pl.pallas_call(
        functools.partial(_m1_body, dh2=h2c),
        grid=grid,
        in_specs=[pl.BlockSpec((2, _ROWS, 16), lambda i: (0, i, 0)),
                  _row_spec(d), _full_spec(d, h)],
        out_specs=[pl.BlockSpec((2, _ROWS, h2c), lambda i: (0, i, 0)),
                   _row_spec(1)],
        out_shape=[
            jax.ShapeDtypeStruct((2, n, h2c), jnp.float32),
            jax.ShapeDtypeStruct((n, 1), jnp.float32),
        ],
    )(deg_parts, x, W1)

    p = _make_mp(n, e2, h2c)(g1.reshape(2 * n, h2c), idx_mp, zeros_h)

    w2p = jnp.pad(W2, ((0, 0), (0, cpad - c)))
    b1r = b1.reshape(1, h)
    g2 = pl.pallas_call(
        functools.partial(_m2_body, dh2=c2c),
        grid=grid,
        in_specs=[_part_spec(0, h2c), _part_spec(1, h2c),
                  pl.BlockSpec((1, _ROWS, h2c), lambda i: (0, i, 0)),
                  pl.BlockSpec((1, _ROWS, h2c), lambda i: (1, i, 0)),
                  _row_spec(1), _full_spec(1, h), _full_spec(h, cpad)],
        out_specs=pl.BlockSpec((2, _ROWS, c2c), lambda i: (0, i, 0)),
        out_shape=jax.ShapeDtypeStruct((2, n, c2c), jnp.float32),
    )(p, p, g1, g1, dinv, b1r, w2p)

    q = _make_mp(n, e2, c2c, spmem_table=True)(g2.reshape(2 * n, c2c), idx_mp, zeros_c)

    b2p = jnp.pad(b2, (0, cpad - c)).reshape(1, cpad)
    out = pl.pallas_call(
        functools.partial(_fin_body, c=c),
        grid=grid,
        in_specs=[_part_spec(0, c2c), _part_spec(1, c2c),
                  pl.BlockSpec((1, _ROWS, c2c), lambda i: (0, i, 0)),
                  pl.BlockSpec((1, _ROWS, c2c), lambda i: (1, i, 0)),
                  _row_spec(1), _full_spec(1, cpad)],
        out_specs=_row_spec(c),
        out_shape=jax.ShapeDtypeStruct((n, c), jnp.float32),
    )(q, q, g2, g2, dinv, b2p)
    return out
